# baseline (speedup 1.0000x reference)
import functools

import jax
import jax.numpy as jnp
from jax import lax
from jax.experimental import pallas as pl
from jax.experimental.pallas import tpu as pltpu
from jax.experimental.pallas import tpu_sc as plsc

_E = 320000
_N = 10000
_H = 16
_NC = 2
_NS = 16
_NW = _NC * _NS
_IR = _E // 128
_RW = 78
_TL = _IR - _NW * _RW
_GR = 13
_NG = _RW // _GR
_GE = _GR * 128
_GP = _GE // 8
_WR = 1000
_EP = _E // 8


@functools.lru_cache(maxsize=None)
def _sc_kernels():
    mesh = plsc.VectorSubcoreMesh(core_axis_name="c", subcore_axis_name="s")

    def _pack(rows_e, rows_p, nrows):
        def body(r, carry):
            for l in range(8):
                v = rows_e[8 * r + l, :]
                rows_p[r, pl.ds(l * _H, _H)] = v
            return carry
        lax.fori_loop(0, nrows, body, 0)

    def _unpack(rows_p, rows_e, nrows):
        def body(r, carry):
            for l in range(8):
                v = rows_p[r, pl.ds(l * _H, _H)]
                rows_e[8 * r + l, :] = v
            return carry
        lax.fori_loop(0, nrows, body, 0)

    @functools.partial(
        pl.kernel,
        mesh=mesh,
        out_type=jax.ShapeDtypeStruct((_EP, 128), jnp.float32),
        scratch_types=[
            pltpu.VMEM((_RW * 128,), jnp.int32),
            pltpu.VMEM((128,), jnp.int32),
            pltpu.VMEM((_GE, _H), jnp.float32),
            pltpu.VMEM((_GP, 128), jnp.float32),
            pltpu.SemaphoreType.DMA,
        ],
        compiler_params=pltpu.CompilerParams(use_tc_tiling_on_sc=False),
    )
    def _sc_gather(node_hbm, idx_hbm, x_hbm, idx_v, idx_t, rows_e, rows_p, sem):
        wid = lax.axis_index("s") * _NC + lax.axis_index("c")
        r0 = wid * _RW
        p0 = r0 * _H
        pltpu.sync_copy(idx_hbm.at[pl.ds(r0 * 128, _RW * 128)], idx_v)

        def body(g, carry):
            descs = [
                pltpu.async_copy(
                    node_hbm.at[idx_v.at[pl.ds((g * _GR + k) * 128, 128)]],
                    rows_e.at[pl.ds(k * 128, 128)], sem)
                for k in range(_GR)
            ]
            for d in descs:
                d.wait()
            _pack(rows_e, rows_p, _GP)
            pltpu.sync_copy(rows_p, x_hbm.at[pl.ds(p0 + g * _GP, _GP)])
            return carry

        lax.fori_loop(0, _NG, body, 0)

        @pl.when(wid < _TL)
        def _():
            tr = _NW * _RW + wid
            pltpu.sync_copy(idx_hbm.at[pl.ds(tr * 128, 128)], idx_t)
            pltpu.async_copy(node_hbm.at[idx_t],
                             rows_e.at[pl.ds(0, 128)], sem).wait()
            _pack(rows_e, rows_p, _H)
            pltpu.sync_copy(rows_p.at[pl.ds(0, _H)],
                            x_hbm.at[pl.ds(tr * _H, _H)])

    @functools.partial(
        pl.kernel,
        mesh=mesh,
        out_type=jax.ShapeDtypeStruct((_NC, _N // 8, 128), jnp.float32),
        scratch_types=[
            pltpu.VMEM_SHARED((_N, _H), jnp.float32),
            pltpu.VMEM((_RW * 128,), jnp.int32),
            pltpu.VMEM((128,), jnp.int32),
            pltpu.VMEM((_GE, _H), jnp.float32),
            pltpu.VMEM((_GP, 128), jnp.float32),
        ],
        compiler_params=pltpu.CompilerParams(use_tc_tiling_on_sc=False),
    )
    def _sc_scatter(msg_hbm, dom_hbm, part_hbm, acc, idx_v, idx_t, rows_e,
                    rows_p):
        cid = lax.axis_index("c")
        sid = lax.axis_index("s")
        wid = sid * _NC + cid
        r0 = wid * _RW
        p0 = r0 * _H

        def zbody(i, carry):
            rows_e[i, :] = jnp.zeros((_H,), jnp.float32)
            return carry

        lax.fori_loop(0, _WR, zbody, 0)

        @pl.when(sid < _N // _WR)
        def _():
            pltpu.sync_copy(rows_e.at[pl.ds(0, _WR)],
                            acc.at[pl.ds(sid * _WR, _WR)])

        plsc.subcore_barrier()
        pltpu.sync_copy(dom_hbm.at[pl.ds(r0 * 128, _RW * 128)], idx_v)

        def body(g, carry):
            pltpu.sync_copy(msg_hbm.at[pl.ds(p0 + g * _GP, _GP)], rows_p)
            _unpack(rows_p, rows_e, _GP)
            for k in range(_GR):
                pltpu.sync_copy(
                    rows_e.at[pl.ds(k * 128, 128)],
                    acc.at[idx_v.at[pl.ds((g * _GR + k) * 128, 128)]],
                    add=True)
            return carry

        lax.fori_loop(0, _NG, body, 0)

        @pl.when(wid < _TL)
        def _():
            tr = _NW * _RW + wid
            pltpu.sync_copy(dom_hbm.at[pl.ds(tr * 128, 128)], idx_t)
            pltpu.sync_copy(msg_hbm.at[pl.ds(tr * _H, _H)],
                            rows_p.at[pl.ds(0, _H)])
            _unpack(rows_p, rows_e, _H)
            pltpu.sync_copy(rows_e.at[pl.ds(0, 128)],
                            acc.at[idx_t], add=True)

        plsc.subcore_barrier()

        @pl.when(sid < _N // _WR)
        def _():
            pltpu.sync_copy(acc.at[pl.ds(sid * _WR, _WR)],
                            rows_e.at[pl.ds(0, _WR)])
            _pack(rows_e, rows_p, _WR // 8)
            pltpu.sync_copy(rows_p.at[pl.ds(0, _WR // 8)],
                            part_hbm.at[cid, pl.ds(sid * (_WR // 8),
                                                   _WR // 8)])

    return _sc_gather, _sc_scatter


_TE = 6400


def _dense_body(ef_ref, xs_ref, wm_ref, b_ref, v_ref, s_ref, o_ref):
    bf = jnp.bfloat16
    f32 = jnp.float32
    mlp8 = jnp.maximum(
        jnp.dot(ef_ref[...].astype(bf), wm_ref[...],
                preferred_element_type=f32) + b_ref[...], 0.0)
    mlpb = mlp8.astype(bf)
    x8b = xs_ref[...].astype(bf)
    acc = jnp.dot(mlpb, s_ref[0], preferred_element_type=f32) * \
        jnp.dot(x8b, v_ref[0], preferred_element_type=f32)
    for k in range(1, _H):
        mk = jnp.dot(mlpb, s_ref[k], preferred_element_type=f32)
        gk = jnp.dot(x8b, v_ref[k], preferred_element_type=f32)
        acc = acc + mk * gk
    o_ref[...] = acc


def _tc_dense(ef8, xs8, Wbd, b8, V, S, interpret=False):
    tp = _TE // 8
    return pl.pallas_call(
        _dense_body,
        grid=(_E // _TE,),
        in_specs=[
            pl.BlockSpec((tp, 128), lambda i: (i, 0)),
            pl.BlockSpec((tp, 128), lambda i: (i, 0)),
            pl.BlockSpec((128, 128), lambda i: (0, 0)),
            pl.BlockSpec((1, 128), lambda i: (0, 0)),
            pl.BlockSpec((_H, 128, 128), lambda i: (0, 0, 0)),
            pl.BlockSpec((_H, 128, 128), lambda i: (0, 0, 0)),
        ],
        out_specs=pl.BlockSpec((tp, 128), lambda i: (i, 0)),
        out_shape=jax.ShapeDtypeStruct((_EP, 128), jnp.float32),
        interpret=interpret,
    )(ef8, xs8, Wbd, b8, V, S)


def _combine_body(p_ref, o_ref):
    o_ref[...] = p_ref[0] + p_ref[1]


def _tc_combine(parts, interpret=False):
    return pl.pallas_call(
        _combine_body,
        out_shape=jax.ShapeDtypeStruct((_N // 8, 128), jnp.float32),
        interpret=interpret,
    )(parts)


def kernel(node_features, edge_features, edge_domain, edge_range, W_mlp, b_mlp, W_E):
    f32 = jnp.float32
    eye8 = jnp.eye(8, dtype=f32)
    eye = jnp.eye(_H, dtype=f32)
    bf = jnp.bfloat16
    ef8 = edge_features.reshape(_EP, 128)
    anchor = ef8[0, 0] * 0.0
    Wbd = (jnp.kron(eye8, W_mlp.T) + anchor).astype(bf)
    b8 = jnp.tile(b_mlp, (8,)).reshape(1, 128) + anchor
    WE = W_E.reshape(_H, _H, _H) + anchor
    V = jnp.stack([jnp.kron(eye8, WE[k].T) for k in range(_H)]).astype(bf)
    S = jnp.stack([
        jnp.kron(eye8, jnp.zeros((_H, _H), f32).at[k].set(1.0))
        for k in range(_H)
    ]).astype(bf)

    sc_gather, sc_scatter = _sc_kernels()
    x8 = sc_gather(node_features, edge_range)
    msg8 = _tc_dense(ef8, x8, Wbd, b8, V, S)
    parts = sc_scatter(msg8, edge_domain)
    return _tc_combine(parts).reshape(_N, _H)


# speedup vs baseline: 1.0554x; 1.0554x over previous
import functools

import jax
import jax.numpy as jnp
from jax import lax
from jax.experimental import pallas as pl
from jax.experimental.pallas import tpu as pltpu
from jax.experimental.pallas import tpu_sc as plsc

_E = 320000
_N = 10000
_H = 16
_NC = 2
_NS = 16
_NW = _NC * _NS
_IR = _E // 128
_RW = 78
_TL = _IR - _NW * _RW
_GR = 13
_NG = _RW // _GR
_GE = _GR * 128
_GP = _GE // 8
_WR = 1000
_EP = _E // 8


@functools.lru_cache(maxsize=None)
def _sc_kernels():
    mesh = plsc.VectorSubcoreMesh(core_axis_name="c", subcore_axis_name="s")

    def _pack(rows_e, rows_p, nrows):
        def body(r, carry):
            for l in range(8):
                v = rows_e[8 * r + l, :]
                rows_p[r, pl.ds(l * _H, _H)] = v
            return carry
        lax.fori_loop(0, nrows, body, 0)

    def _unpack(rows_p, rows_e, nrows):
        def body(r, carry):
            for l in range(8):
                v = rows_p[r, pl.ds(l * _H, _H)]
                rows_e[8 * r + l, :] = v
            return carry
        lax.fori_loop(0, nrows, body, 0)

    @functools.partial(
        pl.kernel,
        mesh=mesh,
        out_type=jax.ShapeDtypeStruct((_EP, 128), jnp.float32),
        scratch_types=[
            pltpu.VMEM((_RW * 128,), jnp.int32),
            pltpu.VMEM((128,), jnp.int32),
            pltpu.VMEM((_GE, _H), jnp.float32),
            pltpu.VMEM((_GP, 128), jnp.float32),
            pltpu.SemaphoreType.DMA,
        ],
        compiler_params=pltpu.CompilerParams(use_tc_tiling_on_sc=False),
    )
    def _sc_gather(node_hbm, idx_hbm, x_hbm, idx_v, idx_t, rows_e, rows_p, sem):
        wid = lax.axis_index("s") * _NC + lax.axis_index("c")
        r0 = wid * _RW
        p0 = r0 * _H
        pltpu.sync_copy(idx_hbm.at[pl.ds(r0 * 128, _RW * 128)], idx_v)

        def body(g, carry):
            descs = [
                pltpu.async_copy(
                    node_hbm.at[idx_v.at[pl.ds((g * _GR + k) * 128, 128)]],
                    rows_e.at[pl.ds(k * 128, 128)], sem)
                for k in range(_GR)
            ]
            for d in descs:
                d.wait()
            _pack(rows_e, rows_p, _GP)
            pltpu.sync_copy(rows_p, x_hbm.at[pl.ds(p0 + g * _GP, _GP)])
            return carry

        lax.fori_loop(0, _NG, body, 0)

        @pl.when(wid < _TL)
        def _():
            tr = _NW * _RW + wid
            pltpu.sync_copy(idx_hbm.at[pl.ds(tr * 128, 128)], idx_t)
            pltpu.async_copy(node_hbm.at[idx_t],
                             rows_e.at[pl.ds(0, 128)], sem).wait()
            _pack(rows_e, rows_p, _H)
            pltpu.sync_copy(rows_p.at[pl.ds(0, _H)],
                            x_hbm.at[pl.ds(tr * _H, _H)])

    @functools.partial(
        pl.kernel,
        mesh=mesh,
        out_type=jax.ShapeDtypeStruct((_NC, _N // 8, 128), jnp.float32),
        scratch_types=[
            pltpu.VMEM_SHARED((_N, _H), jnp.float32),
            pltpu.VMEM((_RW * 128,), jnp.int32),
            pltpu.VMEM((128,), jnp.int32),
            pltpu.VMEM((_GE, _H), jnp.float32),
            pltpu.VMEM((_GP, 128), jnp.float32),
        ],
        compiler_params=pltpu.CompilerParams(use_tc_tiling_on_sc=False),
    )
    def _sc_scatter(msg_hbm, dom_hbm, part_hbm, acc, idx_v, idx_t, rows_e,
                    rows_p):
        cid = lax.axis_index("c")
        sid = lax.axis_index("s")
        wid = sid * _NC + cid
        r0 = wid * _RW
        p0 = r0 * _H

        def zbody(i, carry):
            rows_e[i, :] = jnp.zeros((_H,), jnp.float32)
            return carry

        lax.fori_loop(0, _WR, zbody, 0)

        @pl.when(sid < _N // _WR)
        def _():
            pltpu.sync_copy(rows_e.at[pl.ds(0, _WR)],
                            acc.at[pl.ds(sid * _WR, _WR)])

        plsc.subcore_barrier()
        pltpu.sync_copy(dom_hbm.at[pl.ds(r0 * 128, _RW * 128)], idx_v)

        def body(g, carry):
            pltpu.sync_copy(msg_hbm.at[pl.ds(p0 + g * _GP, _GP)], rows_p)
            _unpack(rows_p, rows_e, _GP)
            for k in range(_GR):
                pltpu.sync_copy(
                    rows_e.at[pl.ds(k * 128, 128)],
                    acc.at[idx_v.at[pl.ds((g * _GR + k) * 128, 128)]],
                    add=True)
            return carry

        lax.fori_loop(0, _NG, body, 0)

        @pl.when(wid < _TL)
        def _():
            tr = _NW * _RW + wid
            pltpu.sync_copy(dom_hbm.at[pl.ds(tr * 128, 128)], idx_t)
            pltpu.sync_copy(msg_hbm.at[pl.ds(tr * _H, _H)],
                            rows_p.at[pl.ds(0, _H)])
            _unpack(rows_p, rows_e, _H)
            pltpu.sync_copy(rows_e.at[pl.ds(0, 128)],
                            acc.at[idx_t], add=True)

        plsc.subcore_barrier()

        @pl.when(sid < _N // _WR)
        def _():
            pltpu.sync_copy(acc.at[pl.ds(sid * _WR, _WR)],
                            rows_e.at[pl.ds(0, _WR)])
            _pack(rows_e, rows_p, _WR // 8)
            pltpu.sync_copy(rows_p.at[pl.ds(0, _WR // 8)],
                            part_hbm.at[cid, pl.ds(sid * (_WR // 8),
                                                   _WR // 8)])

    return _sc_gather, _sc_scatter


_TE = 6400


def _dense_body(ef_ref, xs_ref, wm_ref, b_ref, v_ref, s_ref, o_ref):
    bf = jnp.bfloat16
    f32 = jnp.float32
    mlp8 = jnp.maximum(
        jnp.dot(ef_ref[...].astype(bf), wm_ref[...],
                preferred_element_type=f32) + b_ref[...], 0.0)
    mlpb = mlp8.astype(bf)
    x8b = xs_ref[...].astype(bf)
    acc = jnp.dot(mlpb, s_ref[0], preferred_element_type=f32) * \
        jnp.dot(x8b, v_ref[0], preferred_element_type=f32)
    for k in range(1, _H):
        mk = jnp.dot(mlpb, s_ref[k], preferred_element_type=f32)
        gk = jnp.dot(x8b, v_ref[k], preferred_element_type=f32)
        acc = acc + mk * gk
    o_ref[...] = acc


def _tc_dense(ef8, xs8, Wbd, b8, V, S, interpret=False):
    tp = _TE // 8
    return pl.pallas_call(
        _dense_body,
        grid=(_E // _TE,),
        in_specs=[
            pl.BlockSpec((tp, 128), lambda i: (i, 0)),
            pl.BlockSpec((tp, 128), lambda i: (i, 0)),
            pl.BlockSpec((128, 128), lambda i: (0, 0)),
            pl.BlockSpec((1, 128), lambda i: (0, 0)),
            pl.BlockSpec((_H, 128, 128), lambda i: (0, 0, 0)),
            pl.BlockSpec((_H, 128, 128), lambda i: (0, 0, 0)),
        ],
        out_specs=pl.BlockSpec((tp, 128), lambda i: (i, 0)),
        out_shape=jax.ShapeDtypeStruct((_EP, 128), jnp.float32),
        interpret=interpret,
    )(ef8, xs8, Wbd, b8, V, S)


def _combine_body(p_ref, o_ref):
    o_ref[...] = p_ref[0] + p_ref[1]


def _tc_combine(parts, interpret=False):
    return pl.pallas_call(
        _combine_body,
        out_shape=jax.ShapeDtypeStruct((_N // 8, 128), jnp.float32),
        interpret=interpret,
    )(parts)


def kernel(node_features, edge_features, edge_domain, edge_range, W_mlp, b_mlp, W_E):
    f32 = jnp.float32
    eye8 = jnp.eye(8, dtype=f32)
    eye = jnp.eye(_H, dtype=f32)
    bf = jnp.bfloat16
    Wbd = jnp.kron(eye8, W_mlp.T).astype(bf)
    b8 = jnp.tile(b_mlp, (8,)).reshape(1, 128)
    WE = W_E.reshape(_H, _H, _H)
    V = jnp.stack([jnp.kron(eye8, WE[k].T) for k in range(_H)]).astype(bf)
    S = jnp.stack([
        jnp.kron(eye8, jnp.zeros((_H, _H), f32).at[k].set(1.0))
        for k in range(_H)
    ]).astype(bf)

    sc_gather, sc_scatter = _sc_kernels()
    x8 = sc_gather(node_features, edge_range)
    ef8 = edge_features.reshape(_EP, 128)
    msg8 = _tc_dense(ef8, x8, Wbd, b8, V, S)
    parts = sc_scatter(msg8, edge_domain)
    return _tc_combine(parts).reshape(_N, _H)
